# baseline (speedup 1.0000x reference)
import functools

import jax
import jax.numpy as jnp
from jax import lax
from jax.experimental import pallas as pl
from jax.experimental.pallas import tpu as pltpu
from jax.experimental.pallas import tpu_sc as plsc
import numpy as np

N = 10000
E = 160000
F = 128
FS = 512
NB = 8
G = 8
RMAX = 5.0
AVG = 16.0

NCORES = 2
NSUB = 16
NW = NCORES * NSUB
CHUNK = 128
NCH = 40
EP = NW * NCH * CHUNK
EB = 2048
NBLK = 1000
NP = 10240
NPT = NP // NSUB
SQ3 = float(np.sqrt(3.0))
BESC = float(np.sqrt(2.0 / RMAX))


def _silu(z):
    s = jax.nn.sigmoid(z)
    return z * s


def _dsilu(z):
    s = jax.nn.sigmoid(z)
    return s * (1.0 + z * (1.0 - s))


def _geometry(ps, pr):
    vec = pr - ps
    r2 = jnp.sum(vec * vec, axis=1, keepdims=True) + 1e-12
    r = jnp.sqrt(r2)
    u = vec / r
    x = r / RMAX
    x2 = x * x
    x4 = x2 * x2
    x5 = x4 * x
    x6 = x5 * x
    x7 = x6 * x
    inside = x < 1.0
    env = jnp.where(inside, 1.0 - 21.0 * x5 + 35.0 * x6 - 15.0 * x7, 0.0)
    denv_dr = jnp.where(inside, -105.0 * x4 * (1.0 - x) * (1.0 - x), 0.0) * (1.0 / RMAX)
    narr = lax.broadcasted_iota(jnp.int32, (ps.shape[0], NB), 1).astype(jnp.float32) + 1.0
    warr = narr * (np.pi / RMAX)
    arg = r * warr
    bes = BESC * jnp.sin(arg) / r
    ef = bes * env
    return vec, r, u, env, denv_dr, warr, arg, bes, ef


def _radial_fwd(ef, Wr1, Wr2, Wr3):
    z1 = jnp.dot(ef, Wr1, preferred_element_type=jnp.float32)
    a1 = _silu(z1)
    z2 = jnp.dot(a1, Wr2, preferred_element_type=jnp.float32)
    a2 = _silu(z2)
    rwp = jnp.dot(a2, Wr3, preferred_element_type=jnp.float32)
    return z1, z2, rwp


def _node_pre_body(na_ref, wemb_ref, wmsg0_ref, wsc0_ref, hw0_ref, hsc0_ref):
    na = na_ref[...]
    h0 = jnp.dot(na, wemb_ref[...], preferred_element_type=jnp.float32)
    hw0_ref[...] = jnp.dot(h0, wmsg0_ref[...], preferred_element_type=jnp.float32)
    hsc0_ref[...] = jnp.dot(h0, wsc0_ref[...], preferred_element_type=jnp.float32)


def _edge_fwd_body(ps_ref, pr_ref, hs0_ref, wr1_ref, wr2_ref, wr3_ref, m0_ref):
    ps = ps_ref[...]
    pr = pr_ref[...]
    _, _, u, env, _, _, _, _, ef = _geometry(ps, pr)
    _, _, rwp = _radial_fwd(ef, wr1_ref[...], wr2_ref[...], wr3_ref[...])
    eid = lax.broadcasted_iota(jnp.int32, (EB, F), 0) + pl.program_id(0) * EB
    valid = (eid < E).astype(jnp.float32)
    rw = rwp * env * valid
    t0 = hs0_ref[...] * rw
    m0_ref[0, :, :] = t0
    for k in range(1, 4):
        m0_ref[k, :, :] = t0 * (SQ3 * u[:, k - 1:k])


def _node_mid_body(agg_ref, wp0_ref, hsc0_ref, wmsg1_ref, wsc1_ref, wread0_ref,
                   hw1_ref, hsc1_ref, out0_ref):
    h1 = hsc0_ref[...]
    for k in range(4):
        aggk = (agg_ref[k, 0, :, :] + agg_ref[k, 1, :, :]) * (1.0 / AVG)
        h1 = h1 + jnp.dot(aggk, wp0_ref[k, :, :], preferred_element_type=jnp.float32)
    hw1_ref[...] = jnp.dot(h1, wmsg1_ref[...], preferred_element_type=jnp.float32)
    hsc1_ref[...] = jnp.dot(h1, wsc1_ref[...], preferred_element_type=jnp.float32)
    out0_ref[...] = jnp.dot(h1, wread0_ref[...], preferred_element_type=jnp.float32)


def _edge_msg1_body(ps_ref, pr_ref, hs1_ref, wr1_ref, wr2_ref, wr3_ref, g1t_ref,
                    m1_ref, ghs1_ref):
    ps = ps_ref[...]
    pr = pr_ref[...]
    _, _, u, env, _, _, _, _, ef = _geometry(ps, pr)
    _, _, rwp = _radial_fwd(ef, wr1_ref[...], wr2_ref[...], wr3_ref[...])
    eid = lax.broadcasted_iota(jnp.int32, (EB, F), 0) + pl.program_id(0) * EB
    valid = (eid < E).astype(jnp.float32)
    rw = rwp * env * valid
    hs1 = hs1_ref[...]
    t1 = hs1 * rw
    g1t = g1t_ref[...]
    m1_ref[0, :, :] = t1
    g_t1 = g1t[0:1, :]
    for k in range(1, 4):
        shk = SQ3 * u[:, k - 1:k]
        m1_ref[k, :, :] = t1 * shk
        g_t1 = g_t1 + shk * g1t[k:k + 1, :]
    ghs1_ref[...] = g_t1 * rw


def _node_final_body(agg_ref, wp1_ref, hsc1_ref, ghw_ref, wmsg1t_ref, wp0t_ref,
                     wread1_ref, g1c_ref, out1_ref, ga_ref):
    h2 = hsc1_ref[...]
    for k in range(4):
        aggk = (agg_ref[k, 0, :, :] + agg_ref[k, 1, :, :]) * (1.0 / AVG)
        h2 = h2 + jnp.dot(aggk, wp1_ref[k, :, :], preferred_element_type=jnp.float32)
    out1_ref[...] = jnp.dot(h2, wread1_ref[...], preferred_element_type=jnp.float32)
    ghw1 = ghw_ref[0, 0, :, :] + ghw_ref[0, 1, :, :]
    g_h1 = g1c_ref[...] + jnp.dot(ghw1, wmsg1t_ref[...], preferred_element_type=jnp.float32)
    for k in range(4):
        ga_ref[k, :, :] = jnp.dot(g_h1, wp0t_ref[k, :, :],
                                  preferred_element_type=jnp.float32) * (1.0 / AVG)


def _edge_bwd_body(ps_ref, pr_ref, hs0_ref, hs1_ref,
                   ge0_ref, ge1_ref, ge2_ref, ge3_ref,
                   wr10_ref, wr20_ref, wr30_ref, wr11_ref, wr21_ref, wr31_ref,
                   wr10t_ref, wr20t_ref, wr30t_ref, wr11t_ref, wr21t_ref, wr31t_ref,
                   g1t_ref, gv_ref):
    ps = ps_ref[...]
    pr = pr_ref[...]
    _, r, u, env, denv_dr, warr, arg, bes, ef = _geometry(ps, pr)
    eidv = lax.broadcasted_iota(jnp.int32, (ps.shape[0], 16), 0) + pl.program_id(0) * ps.shape[0]
    valid16 = (eidv < E).astype(jnp.float32)

    z1_0, z2_0, rwp0 = _radial_fwd(ef, wr10_ref[...], wr20_ref[...], wr30_ref[...])
    z1_1, z2_1, rwp1 = _radial_fwd(ef, wr11_ref[...], wr21_ref[...], wr31_ref[...])
    rw0 = rwp0 * env
    rw1 = rwp1 * env
    hs0 = hs0_ref[...]
    hs1 = hs1_ref[...]
    t0 = hs0 * rw0
    t1 = hs1 * rw1
    g1t = g1t_ref[...]

    g_t1 = g1t[0:1, :]
    for k in range(1, 4):
        g_t1 = g_t1 + (SQ3 * u[:, k - 1:k]) * g1t[k:k + 1, :]
    g_rw1 = g_t1 * hs1
    g_sh = [jnp.sum(t1 * g1t[k:k + 1, :], axis=1, keepdims=True) for k in range(4)]

    ge = [ge0_ref[...], ge1_ref[...], ge2_ref[...], ge3_ref[...]]
    g_t0 = ge[0]
    g_sh[0] = g_sh[0] + jnp.sum(ge[0] * t0, axis=1, keepdims=True)
    for k in range(1, 4):
        g_t0 = g_t0 + ge[k] * (SQ3 * u[:, k - 1:k])
        g_sh[k] = g_sh[k] + jnp.sum(ge[k] * t0, axis=1, keepdims=True)
    g_rw0 = g_t0 * hs0

    def radial_bwd(g_rw, rwp, z1, z2, w3t, w2t, w1t):
        g_cut = jnp.sum(g_rw * rwp, axis=1, keepdims=True)
        g_rwp = g_rw * env
        g_a2 = jnp.dot(g_rwp, w3t, preferred_element_type=jnp.float32)
        g_z2 = g_a2 * _dsilu(z2)
        g_a1 = jnp.dot(g_z2, w2t, preferred_element_type=jnp.float32)
        g_z1 = g_a1 * _dsilu(z1)
        g_ef = jnp.dot(g_z1, w1t, preferred_element_type=jnp.float32)
        return g_ef, g_cut

    g_ef0, g_cut0 = radial_bwd(g_rw0, rwp0, z1_0, z2_0, wr30t_ref[...], wr20t_ref[...], wr10t_ref[...])
    g_ef1, g_cut1 = radial_bwd(g_rw1, rwp1, z1_1, z2_1, wr31t_ref[...], wr21t_ref[...], wr11t_ref[...])
    g_ef = g_ef0 + g_ef1
    g_cut = g_cut0 + g_cut1

    g_env = g_cut + jnp.sum(g_ef * bes, axis=1, keepdims=True)
    g_bes = g_ef * env
    dbes_dr = (BESC * warr * jnp.cos(arg) - bes) / r
    g_r = jnp.sum(g_bes * dbes_dr, axis=1, keepdims=True) + g_env * denv_dr

    lane = lax.broadcasted_iota(jnp.int32, (ps.shape[0], 16), 1)
    gv = jnp.zeros(ps.shape, jnp.float32)
    udot = jnp.zeros((ps.shape[0], 1), jnp.float32)
    g_u = [None, None, None]
    for k in range(3):
        g_u[k] = SQ3 * g_sh[k + 1]
        udot = udot + u[:, k:k + 1] * g_u[k]
    for k in range(3):
        uk = u[:, k:k + 1]
        gvk = g_u[k] / r - uk * udot / r + uk * g_r
        gv = gv + gvk * (lane == k).astype(jnp.float32)
    gv_ref[...] = gv * valid16


def _finish_body(na_ref, e0p_ref, out0_ref, out1_ref, ch_ref, pos_ref, batch_ref,
                 fr_ref, fs_ref, forces_ref, ad_ref, ge_ref, gd_ref, gb_ref):
    pid = pl.program_id(0)
    out0 = out0_ref[...]
    out1 = out1_ref[...]
    outs = out0 + out1
    ad_ref[...] = outs
    forces_ref[...] = -(fr_ref[0, 0, :, :] + fr_ref[0, 1, :, :]
                        - fs_ref[0, 0, :, :] - fs_ref[0, 1, :, :])
    ne0 = jnp.dot(na_ref[...], e0p_ref[...], preferred_element_type=jnp.float32)
    lane8 = lax.broadcasted_iota(jnp.int32, (NBLK, 8), 1)
    l0 = (lane8 == 0).astype(jnp.float32)
    epn = ne0 + outs * l0
    onehot = (batch_ref[...] == lane8).astype(jnp.float32)
    cp = ch_ref[...] * pos_ref[...]
    dn = (((0,), (0,)), ((), ()))
    gE = lax.dot_general(onehot, epn, dn, preferred_element_type=jnp.float32)
    gD = lax.dot_general(onehot, outs, dn, preferred_element_type=jnp.float32)
    gB = lax.dot_general(onehot, cp, dn, preferred_element_type=jnp.float32)

    @pl.when(pid == 0)
    def _():
        ge_ref[...] = gE
        gd_ref[...] = gD
        gb_ref[...] = gB

    @pl.when(pid != 0)
    def _():
        ge_ref[...] = ge_ref[...] + gE
        gd_ref[...] = gd_ref[...] + gD
        gb_ref[...] = gb_ref[...] + gB


def _sc_gather_multi(pairs, D):
    P = len(pairs)
    mesh = plsc.VectorSubcoreMesh(core_axis_name="c", subcore_axis_name="s")

    @functools.partial(
        pl.kernel, mesh=mesh,
        compiler_params=pltpu.CompilerParams(use_tc_tiling_on_sc=(D % 128 == 0)),
        out_type=[jax.ShapeDtypeStruct((EP, D), jnp.float32) for _ in range(P)],
        scratch_types=(
            [pltpu.VMEM((NCH, CHUNK), jnp.int32)]
            + [pltpu.VMEM((CHUNK, D), jnp.float32) for _ in range(4)]
            + [pltpu.SemaphoreType.DMA for _ in range(8)]
        ),
    )
    def k(*args):
        tables = args[:P]
        idxs = args[P:2 * P]
        outs = args[2 * P:3 * P]
        idx_v = args[3 * P]
        bufs = args[3 * P + 1:3 * P + 5]
        gs = args[3 * P + 5:3 * P + 9]
        ws = args[3 * P + 9:3 * P + 13]
        c = lax.axis_index("c")
        s = lax.axis_index("s")
        wid = c * NSUB + s

        for p in range(P):
            table, out = tables[p], outs[p]
            pltpu.sync_copy(idxs[p].at[c, s], idx_v)

            def gat_start(j, b):
                pltpu.async_copy(table.at[idx_v.at[j]], bufs[b], gs[b])

            def gat_wait(b):
                pltpu.make_async_copy(table.at[idx_v.at[0]], bufs[b], gs[b]).wait()

            def wb_start(j, b):
                base = (wid * NCH + j) * CHUNK
                pltpu.async_copy(bufs[b], out.at[pl.ds(base, CHUNK)], ws[b])

            def wb_wait(b):
                base = wid * NCH * CHUNK
                pltpu.make_async_copy(bufs[b], out.at[pl.ds(base, CHUNK)], ws[b]).wait()

            for b in range(4):
                gat_start(b, b)

            def body(it, _):
                j = it * 4
                for b in range(4):
                    gat_wait(b)
                    wb_start(j + b, b)
                for b in range(4):
                    wb_wait(b)
                    gat_start(j + 4 + b, b)
                return 0

            lax.fori_loop(0, NCH // 4 - 1, body, 0)
            jl = NCH - 4
            for b in range(4):
                gat_wait(b)
                wb_start(jl + b, b)
            for b in range(4):
                wb_wait(b)

    ins = [p[0] for p in pairs] + [p[1] for p in pairs]
    return list(k(*ins))


def _sc_scatter(data, idx4, K, D):
    mesh = plsc.VectorSubcoreMesh(core_axis_name="c", subcore_axis_name="s")
    zeros = jnp.zeros((NPT, D), jnp.float32)

    @functools.partial(
        pl.kernel, mesh=mesh,
        compiler_params=pltpu.CompilerParams(use_tc_tiling_on_sc=(D % 128 == 0)),
        out_type=jax.ShapeDtypeStruct((K, NCORES, NP, D), jnp.float32),
        scratch_types=(
            [pltpu.VMEM((NCH, CHUNK), jnp.int32)]
            + [pltpu.VMEM((CHUNK, D), jnp.float32) for _ in range(2)]
            + [pltpu.SemaphoreType.DMA for _ in range(4)]
            + [pltpu.VMEM_SHARED((NP, D), jnp.float32)]
        ),
    )
    def k(data_hbm, idx_hbm, z_hbm, out_hbm, idx_v, b0, b1,
          r0, r1, t0, t1, acc_sh):
        bufs = (b0, b1)
        rs = (r0, r1)
        ss = (t0, t1)
        c = lax.axis_index("c")
        s = lax.axis_index("s")
        pltpu.sync_copy(idx_hbm.at[c, s], idx_v)
        base_rows = (c * NSUB + s) * NCH * CHUNK
        for kk in range(K):
            pltpu.sync_copy(z_hbm, acc_sh.at[pl.ds(s * NPT, NPT)])
            plsc.subcore_barrier()

            def rd_start(j, b):
                pltpu.async_copy(data_hbm.at[kk, pl.ds(base_rows + j * CHUNK, CHUNK)], bufs[b], rs[b])

            def rd_wait(b):
                pltpu.make_async_copy(data_hbm.at[kk, pl.ds(base_rows, CHUNK)], bufs[b], rs[b]).wait()

            def scat_start(j, b):
                pltpu.async_copy(bufs[b], acc_sh.at[idx_v.at[j]], ss[b], add=True)

            def scat_wait(b):
                pltpu.make_async_copy(bufs[b], acc_sh.at[idx_v.at[0]], ss[b]).wait()

            for b in range(2):
                rd_start(b, b)

            def body(it, _):
                j = it * 2
                for b in range(2):
                    rd_wait(b)
                    scat_start(j + b, b)
                for b in range(2):
                    scat_wait(b)
                    rd_start(j + 2 + b, b)
                return 0

            lax.fori_loop(0, NCH // 2 - 1, body, 0)
            jl = NCH - 2
            for b in range(2):
                rd_wait(b)
                scat_start(jl + b, b)
            for b in range(2):
                scat_wait(b)
            plsc.subcore_barrier()
            pltpu.sync_copy(acc_sh.at[pl.ds(s * NPT, NPT)],
                            out_hbm.at[kk, c, pl.ds(s * NPT, NPT)])
            plsc.subcore_barrier()

    return k(data, idx4, zeros)


def _sc_scatter_forces(data, ridx4, sidx4):
    mesh = plsc.VectorSubcoreMesh(core_axis_name="c", subcore_axis_name="s")
    zeros = jnp.zeros((NPT, 16), jnp.float32)

    @functools.partial(
        pl.kernel, mesh=mesh,
        compiler_params=pltpu.CompilerParams(use_tc_tiling_on_sc=False),
        out_type=jax.ShapeDtypeStruct((2, NCORES, NP, 16), jnp.float32),
        scratch_types=[
            pltpu.VMEM((NCH, CHUNK), jnp.int32),
            pltpu.VMEM((NCH, CHUNK), jnp.int32),
            pltpu.VMEM((CHUNK, 16), jnp.float32),
            pltpu.VMEM((CHUNK, 16), jnp.float32),
            pltpu.SemaphoreType.DMA,
            pltpu.SemaphoreType.DMA,
            pltpu.VMEM_SHARED((NP, 16), jnp.float32),
            pltpu.VMEM_SHARED((NP, 16), jnp.float32),
        ],
    )
    def k(data_hbm, ridx_hbm, sidx_hbm, z_hbm, out_hbm,
          ridx_v, sidx_v, buf0, buf1, rs0, rs1, accR, accS):
        c = lax.axis_index("c")
        s = lax.axis_index("s")
        pltpu.sync_copy(ridx_hbm.at[c, s], ridx_v)
        pltpu.sync_copy(sidx_hbm.at[c, s], sidx_v)
        base_rows = (c * NSUB + s) * NCH * CHUNK
        pltpu.sync_copy(z_hbm, accR.at[pl.ds(s * NPT, NPT)])
        pltpu.sync_copy(z_hbm, accS.at[pl.ds(s * NPT, NPT)])
        plsc.subcore_barrier()

        def rd_start(j, buf, sem):
            pltpu.async_copy(data_hbm.at[pl.ds(base_rows + j * CHUNK, CHUNK)], buf, sem)

        def rd_wait(buf, sem):
            pltpu.make_async_copy(data_hbm.at[pl.ds(base_rows, CHUNK)], buf, sem).wait()

        def scat(j, buf):
            pltpu.sync_copy(buf, accR.at[ridx_v.at[j]], add=True)
            pltpu.sync_copy(buf, accS.at[sidx_v.at[j]], add=True)

        rd_start(0, buf0, rs0)
        rd_start(1, buf1, rs1)

        def body(it, _):
            j = it * 2
            rd_wait(buf0, rs0)
            scat(j, buf0)
            rd_start(j + 2, buf0, rs0)
            rd_wait(buf1, rs1)
            scat(j + 1, buf1)
            rd_start(j + 3, buf1, rs1)
            return 0

        lax.fori_loop(0, NCH // 2 - 1, body, 0)
        jl = NCH - 2
        rd_wait(buf0, rs0)
        scat(jl, buf0)
        rd_wait(buf1, rs1)
        scat(jl + 1, buf1)
        plsc.subcore_barrier()
        pltpu.sync_copy(accR.at[pl.ds(s * NPT, NPT)], out_hbm.at[0, c, pl.ds(s * NPT, NPT)])
        pltpu.sync_copy(accS.at[pl.ds(s * NPT, NPT)], out_hbm.at[1, c, pl.ds(s * NPT, NPT)])
        plsc.subcore_barrier()

    return k(data, ridx4, sidx4, zeros)


def _full(shape):
    return pl.BlockSpec(shape, lambda i: tuple(0 for _ in shape))


def _tc_node_pre(node_attrs, W_embed, Wmsg_0, Wsc_0, interpret=False):
    grid = (N // NBLK,)
    return pl.pallas_call(
        _node_pre_body,
        grid=grid,
        in_specs=[
            pl.BlockSpec((NBLK, 10), lambda i: (i, 0)),
            _full((10, F)), _full((F, F)), _full((F, FS)),
        ],
        out_specs=[
            pl.BlockSpec((NBLK, F), lambda i: (i, 0)),
            pl.BlockSpec((NBLK, FS), lambda i: (i, 0)),
        ],
        out_shape=[
            jax.ShapeDtypeStruct((N, F), jnp.float32),
            jax.ShapeDtypeStruct((N, FS), jnp.float32),
        ],
        interpret=interpret,
    )(node_attrs, W_embed, Wmsg_0, Wsc_0)


def _tc_edge_fwd(ps, pr, hs0, Wr1, Wr2, Wr3, interpret=False):
    grid = (EP // EB,)
    return pl.pallas_call(
        _edge_fwd_body,
        grid=grid,
        in_specs=[
            pl.BlockSpec((EB, 16), lambda i: (i, 0)),
            pl.BlockSpec((EB, 16), lambda i: (i, 0)),
            pl.BlockSpec((EB, F), lambda i: (i, 0)),
            _full((NB, 64)), _full((64, 64)), _full((64, F)),
        ],
        out_specs=pl.BlockSpec((4, EB, F), lambda i: (0, i, 0)),
        out_shape=jax.ShapeDtypeStruct((4, EP, F), jnp.float32),
        interpret=interpret,
    )(ps, pr, hs0, Wr1, Wr2, Wr3)


def _tc_node_mid(aggP0, Wp0k, hsc0, Wmsg_1, Wsc_1, Wread0p, interpret=False):
    grid = (N // NBLK,)
    return pl.pallas_call(
        _node_mid_body,
        grid=grid,
        in_specs=[
            pl.BlockSpec((4, NCORES, NBLK, F), lambda i: (0, 0, i, 0)),
            _full((4, F, FS)),
            pl.BlockSpec((NBLK, FS), lambda i: (i, 0)),
            _full((FS, F)), _full((FS, FS)), _full((FS, 8)),
        ],
        out_specs=[
            pl.BlockSpec((NBLK, F), lambda i: (i, 0)),
            pl.BlockSpec((NBLK, FS), lambda i: (i, 0)),
            pl.BlockSpec((NBLK, 8), lambda i: (i, 0)),
        ],
        out_shape=[
            jax.ShapeDtypeStruct((N, F), jnp.float32),
            jax.ShapeDtypeStruct((N, FS), jnp.float32),
            jax.ShapeDtypeStruct((N, 8), jnp.float32),
        ],
        interpret=interpret,
    )(aggP0, Wp0k, hsc0, Wmsg_1, Wsc_1, Wread0p)


def _tc_edge_msg1(ps, pr, hs1, Wr1, Wr2, Wr3, G1T, interpret=False):
    grid = (EP // EB,)
    return pl.pallas_call(
        _edge_msg1_body,
        grid=grid,
        in_specs=[
            pl.BlockSpec((EB, 16), lambda i: (i, 0)),
            pl.BlockSpec((EB, 16), lambda i: (i, 0)),
            pl.BlockSpec((EB, F), lambda i: (i, 0)),
            _full((NB, 64)), _full((64, 64)), _full((64, F)), _full((4, F)),
        ],
        out_specs=[
            pl.BlockSpec((4, EB, F), lambda i: (0, i, 0)),
            pl.BlockSpec((EB, F), lambda i: (i, 0)),
        ],
        out_shape=[
            jax.ShapeDtypeStruct((4, EP, F), jnp.float32),
            jax.ShapeDtypeStruct((EP, F), jnp.float32),
        ],
        interpret=interpret,
    )(ps, pr, hs1, Wr1, Wr2, Wr3, G1T)


def _tc_node_final(aggP1, Wp1k, hsc1, ghWP, Wmsg1T, Wp0kT, Wread1p, g1c, interpret=False):
    grid = (N // NBLK,)
    return pl.pallas_call(
        _node_final_body,
        grid=grid,
        in_specs=[
            pl.BlockSpec((4, NCORES, NBLK, F), lambda i: (0, 0, i, 0)),
            _full((4, F, FS)),
            pl.BlockSpec((NBLK, FS), lambda i: (i, 0)),
            pl.BlockSpec((1, NCORES, NBLK, F), lambda i: (0, 0, i, 0)),
            _full((F, FS)), _full((4, FS, F)), _full((FS, 8)), _full((1, FS)),
        ],
        out_specs=[
            pl.BlockSpec((NBLK, 8), lambda i: (i, 0)),
            pl.BlockSpec((4, NBLK, F), lambda i: (0, i, 0)),
        ],
        out_shape=[
            jax.ShapeDtypeStruct((N, 8), jnp.float32),
            jax.ShapeDtypeStruct((4, N, F), jnp.float32),
        ],
        interpret=interpret,
    )(aggP1, Wp1k, hsc1, ghWP, Wmsg1T, Wp0kT, Wread1p, g1c)


def _tc_edge_bwd(ps, pr, hs0, hs1, ge_list, wpack, interpret=False):
    grid = (EP // EB,)
    (Wr10, Wr20, Wr30, Wr11, Wr21, Wr31,
     Wr10T, Wr20T, Wr30T, Wr11T, Wr21T, Wr31T, G1T) = wpack
    return pl.pallas_call(
        _edge_bwd_body,
        grid=grid,
        in_specs=[
            pl.BlockSpec((EB, 16), lambda i: (i, 0)),
            pl.BlockSpec((EB, 16), lambda i: (i, 0)),
            pl.BlockSpec((EB, F), lambda i: (i, 0)),
            pl.BlockSpec((EB, F), lambda i: (i, 0)),
            pl.BlockSpec((EB, F), lambda i: (i, 0)),
            pl.BlockSpec((EB, F), lambda i: (i, 0)),
            pl.BlockSpec((EB, F), lambda i: (i, 0)),
            pl.BlockSpec((EB, F), lambda i: (i, 0)),
            _full((NB, 64)), _full((64, 64)), _full((64, F)),
            _full((NB, 64)), _full((64, 64)), _full((64, F)),
            _full((64, NB)), _full((64, 64)), _full((F, 64)),
            _full((64, NB)), _full((64, 64)), _full((F, 64)),
            _full((4, F)),
        ],
        out_specs=pl.BlockSpec((EB, 16), lambda i: (i, 0)),
        out_shape=jax.ShapeDtypeStruct((EP, 16), jnp.float32),
        interpret=interpret,
    )(ps, pr, hs0, hs1, *ge_list,
      Wr10, Wr20, Wr30, Wr11, Wr21, Wr31,
      Wr10T, Wr20T, Wr30T, Wr11T, Wr21T, Wr31T, G1T)


def _tc_finish(node_attrs, E0p, out0c, out1c, charges2, pos16, batch2, fR, fS,
               interpret=False):
    grid = (N // NBLK,)
    return pl.pallas_call(
        _finish_body,
        grid=grid,
        in_specs=[
            pl.BlockSpec((NBLK, 10), lambda i: (i, 0)),
            _full((10, 8)),
            pl.BlockSpec((NBLK, 8), lambda i: (i, 0)),
            pl.BlockSpec((NBLK, 8), lambda i: (i, 0)),
            pl.BlockSpec((NBLK, 1), lambda i: (i, 0)),
            pl.BlockSpec((NBLK, 16), lambda i: (i, 0)),
            pl.BlockSpec((NBLK, 8), lambda i: (i, 0)),
            pl.BlockSpec((1, NCORES, NBLK, 16), lambda i: (0, 0, i, 0)),
            pl.BlockSpec((1, NCORES, NBLK, 16), lambda i: (0, 0, i, 0)),
        ],
        out_specs=[
            pl.BlockSpec((NBLK, 16), lambda i: (i, 0)),
            pl.BlockSpec((NBLK, 8), lambda i: (i, 0)),
            _full((G, 8)), _full((G, 8)), _full((G, 16)),
        ],
        out_shape=[
            jax.ShapeDtypeStruct((N, 16), jnp.float32),
            jax.ShapeDtypeStruct((N, 8), jnp.float32),
            jax.ShapeDtypeStruct((G, 8), jnp.float32),
            jax.ShapeDtypeStruct((G, 8), jnp.float32),
            jax.ShapeDtypeStruct((G, 16), jnp.float32),
        ],
        interpret=interpret,
    )(node_attrs, E0p, out0c, out1c, charges2, pos16, batch2, fR, fS)


def kernel(node_attrs, positions, shifts, charges, edge_index, batch, head, ptr,
           W_embed, E0, Wr1_0, Wr2_0, Wr3_0, Wmsg_0, Wsc_0, Wprod_0, Wread_0,
           Wr1_1, Wr2_1, Wr3_1, Wmsg_1, Wsc_1, Wprod_1, Wread_1):
    f32 = jnp.float32
    sender = edge_index[0]
    recv = edge_index[1]
    pad = EP - E
    sender_p = jnp.pad(sender, (0, pad)).astype(jnp.int32)
    recv_p = jnp.pad(recv, (0, pad)).astype(jnp.int32)
    sidx4 = sender_p.reshape(NCORES, NSUB, NCH, CHUNK)
    ridx4 = recv_p.reshape(NCORES, NSUB, NCH, CHUNK)
    pos16 = jnp.pad(positions.astype(f32), ((0, 0), (0, 13)))
    charges2 = charges.astype(f32)[:, None]
    batch2 = jnp.broadcast_to(batch.astype(jnp.int32)[:, None], (N, 8))

    Wp0k = Wprod_0.reshape(F, 4, FS).transpose(1, 0, 2)
    Wp1k = Wprod_1.reshape(F, 4, FS).transpose(1, 0, 2)
    Wp0kT = Wp0k.transpose(0, 2, 1)
    Wread0p = jnp.pad(Wread_0, ((0, 0), (0, 4)))
    Wread1p = jnp.pad(Wread_1, ((0, 0), (0, 4)))
    E0p = jnp.pad(E0.T, ((0, 0), (0, 7)))
    G1T = (Wprod_1 @ Wread_1[:, 0]).reshape(F, 4).T / AVG
    g1c = (Wread_0[:, 0] + Wsc_1 @ Wread_1[:, 0])[None, :]
    Wmsg1T = Wmsg_1.T
    wpack = (Wr1_0, Wr2_0, Wr3_0, Wr1_1, Wr2_1, Wr3_1,
             Wr1_0.T, Wr2_0.T, Wr3_0.T, Wr1_1.T, Wr2_1.T, Wr3_1.T, G1T)

    hW0, hsc0 = _tc_node_pre(node_attrs, W_embed, Wmsg_0, Wsc_0)
    ps, pr = _sc_gather_multi([(pos16, sidx4), (pos16, ridx4)], 16)
    hs0, = _sc_gather_multi([(hW0, sidx4)], F)
    m0 = _tc_edge_fwd(ps, pr, hs0, Wr1_0, Wr2_0, Wr3_0)
    aggP0 = _sc_scatter(m0, ridx4, 4, F)
    hW1, hsc1, out0c = _tc_node_mid(aggP0, Wp0k, hsc0, Wmsg_1, Wsc_1, Wread0p)
    hs1, = _sc_gather_multi([(hW1, sidx4)], F)
    m1, ghs1 = _tc_edge_msg1(ps, pr, hs1, Wr1_1, Wr2_1, Wr3_1, G1T)
    aggP1 = _sc_scatter(m1, ridx4, 4, F)
    ghWP = _sc_scatter(ghs1[None], sidx4, 1, F)
    out1c, Ga4 = _tc_node_final(aggP1, Wp1k, hsc1, ghWP, Wmsg1T, Wp0kT, Wread1p, g1c)

    ge_list = _sc_gather_multi([(Ga4[k], ridx4) for k in range(4)], F)
    gv16 = _tc_edge_bwd(ps, pr, hs0, hs1, ge_list, wpack)
    fRS = _sc_scatter_forces(gv16, ridx4, sidx4)
    fR = fRS[0:1]
    fS = fRS[1:2]

    forces16, ad8, gE, gD, gB = _tc_finish(
        node_attrs, E0p, out0c, out1c, charges2, pos16, batch2, fR, fS)
    total_energy = gE[:, 0]
    total_dipole = gD[:, 1:4] + gB[:, 0:3]
    atomic_dipoles = ad8[:, 1:4]
    forces = forces16[:, 0:3]
    return total_energy, forces, total_dipole, atomic_dipoles


# speedup vs baseline: 1.3671x; 1.3671x over previous
import functools

import jax
import jax.numpy as jnp
from jax import lax
from jax.experimental import pallas as pl
from jax.experimental.pallas import tpu as pltpu
from jax.experimental.pallas import tpu_sc as plsc
import numpy as np

N = 10000
E = 160000
F = 128
FS = 512
NB = 8
G = 8
RMAX = 5.0
AVG = 16.0

NCORES = 2
NSUB = 16
NW = NCORES * NSUB
CHUNK = 128
NCH = 40
EP = NW * NCH * CHUNK
EB = 2048
NBLK = 1000
NP = 10240
NPT = NP // NSUB
SQ3 = float(np.sqrt(3.0))
BESC = float(np.sqrt(2.0 / RMAX))


def _silu(z):
    s = jax.nn.sigmoid(z)
    return z * s


def _dsilu(z):
    s = jax.nn.sigmoid(z)
    return s * (1.0 + z * (1.0 - s))


def _geometry(ps, pr):
    vec = pr - ps
    r2 = jnp.sum(vec * vec, axis=1, keepdims=True) + 1e-12
    r = jnp.sqrt(r2)
    u = vec / r
    x = r / RMAX
    x2 = x * x
    x4 = x2 * x2
    x5 = x4 * x
    x6 = x5 * x
    x7 = x6 * x
    inside = x < 1.0
    env = jnp.where(inside, 1.0 - 21.0 * x5 + 35.0 * x6 - 15.0 * x7, 0.0)
    denv_dr = jnp.where(inside, -105.0 * x4 * (1.0 - x) * (1.0 - x), 0.0) * (1.0 / RMAX)
    narr = lax.broadcasted_iota(jnp.int32, (ps.shape[0], NB), 1).astype(jnp.float32) + 1.0
    warr = narr * (np.pi / RMAX)
    arg = r * warr
    bes = BESC * jnp.sin(arg) / r
    ef = bes * env
    return vec, r, u, env, denv_dr, warr, arg, bes, ef


def _radial_fwd(ef, Wr1, Wr2, Wr3):
    z1 = jnp.dot(ef, Wr1, preferred_element_type=jnp.float32)
    a1 = _silu(z1)
    z2 = jnp.dot(a1, Wr2, preferred_element_type=jnp.float32)
    a2 = _silu(z2)
    rwp = jnp.dot(a2, Wr3, preferred_element_type=jnp.float32)
    return z1, z2, rwp


def _node_pre_body(na_ref, wemb_ref, wmsg0_ref, wsc0_ref, hw0_ref, hsc0_ref):
    na = na_ref[...]
    h0 = jnp.dot(na, wemb_ref[...], preferred_element_type=jnp.float32)
    hw0_ref[...] = jnp.dot(h0, wmsg0_ref[...], preferred_element_type=jnp.float32)
    hsc0_ref[...] = jnp.dot(h0, wsc0_ref[...], preferred_element_type=jnp.float32)


def _edge_fwd_body(ps_ref, pr_ref, hs0_ref, wr1_ref, wr2_ref, wr3_ref, m0_ref):
    ps = ps_ref[...]
    pr = pr_ref[...]
    _, _, u, env, _, _, _, _, ef = _geometry(ps, pr)
    _, _, rwp = _radial_fwd(ef, wr1_ref[...], wr2_ref[...], wr3_ref[...])
    eid = lax.broadcasted_iota(jnp.int32, (EB, F), 0) + pl.program_id(0) * EB
    valid = (eid < E).astype(jnp.float32)
    rw = rwp * env * valid
    t0 = hs0_ref[...] * rw
    m0_ref[0, :, :] = t0
    for k in range(1, 4):
        m0_ref[k, :, :] = t0 * (SQ3 * u[:, k - 1:k])


def _node_mid_body(agg_ref, wp0_ref, hsc0_ref, wmsg1_ref, wsc1_ref, wread0_ref,
                   hw1_ref, hsc1_ref, out0_ref):
    h1 = hsc0_ref[...]
    for k in range(4):
        aggk = (agg_ref[k, 0, :, :] + agg_ref[k, 1, :, :]) * (1.0 / AVG)
        h1 = h1 + jnp.dot(aggk, wp0_ref[k, :, :], preferred_element_type=jnp.float32)
    hw1_ref[...] = jnp.dot(h1, wmsg1_ref[...], preferred_element_type=jnp.float32)
    hsc1_ref[...] = jnp.dot(h1, wsc1_ref[...], preferred_element_type=jnp.float32)
    out0_ref[...] = jnp.dot(h1, wread0_ref[...], preferred_element_type=jnp.float32)


def _edge_msg1_body(ps_ref, pr_ref, hs1_ref, wr1_ref, wr2_ref, wr3_ref, g1t_ref,
                    w4_ref, q1_ref, ghs1_ref):
    ps = ps_ref[...]
    pr = pr_ref[...]
    _, _, u, env, _, _, _, _, ef = _geometry(ps, pr)
    _, _, rwp = _radial_fwd(ef, wr1_ref[...], wr2_ref[...], wr3_ref[...])
    eid = lax.broadcasted_iota(jnp.int32, (EB, F), 0) + pl.program_id(0) * EB
    valid = (eid < E).astype(jnp.float32)
    rw = rwp * env * valid
    hs1 = hs1_ref[...]
    t1 = hs1 * rw
    g1t = g1t_ref[...]
    q1 = jnp.dot(t1, w4_ref[0, :, :], preferred_element_type=jnp.float32)
    g_t1 = g1t[0:1, :]
    for k in range(1, 4):
        shk = SQ3 * u[:, k - 1:k]
        q1 = q1 + shk * jnp.dot(t1, w4_ref[k, :, :], preferred_element_type=jnp.float32)
        g_t1 = g_t1 + shk * g1t[k:k + 1, :]
    q1_ref[...] = q1
    ghs1_ref[...] = g_t1 * rw


def _node_gw_body(ghw_ref, gw_ref):
    gw_ref[...] = ghw_ref[0, 0, :, :] + ghw_ref[0, 1, :, :]


def _edge_bwd_body(ps_ref, pr_ref, hs0_ref, hs1_ref, gwv_ref,
                   wr10_ref, wr20_ref, wr30_ref, wr11_ref, wr21_ref, wr31_ref,
                   wr10t_ref, wr20t_ref, wr30t_ref, wr11t_ref, wr21t_ref, wr31t_ref,
                   g1t_ref, m4_ref, c04_ref, gv_ref):
    ps = ps_ref[...]
    pr = pr_ref[...]
    _, r, u, env, denv_dr, warr, arg, bes, ef = _geometry(ps, pr)
    eidv = lax.broadcasted_iota(jnp.int32, (ps.shape[0], 16), 0) + pl.program_id(0) * ps.shape[0]
    valid16 = (eidv < E).astype(jnp.float32)

    z1_0, z2_0, rwp0 = _radial_fwd(ef, wr10_ref[...], wr20_ref[...], wr30_ref[...])
    z1_1, z2_1, rwp1 = _radial_fwd(ef, wr11_ref[...], wr21_ref[...], wr31_ref[...])
    rw0 = rwp0 * env
    rw1 = rwp1 * env
    hs0 = hs0_ref[...]
    hs1 = hs1_ref[...]
    t0 = hs0 * rw0
    t1 = hs1 * rw1
    g1t = g1t_ref[...]

    g_t1 = g1t[0:1, :]
    for k in range(1, 4):
        g_t1 = g_t1 + (SQ3 * u[:, k - 1:k]) * g1t[k:k + 1, :]
    g_rw1 = g_t1 * hs1
    g_sh = [jnp.sum(t1 * g1t[k:k + 1, :], axis=1, keepdims=True) for k in range(4)]

    gwv = gwv_ref[...]
    ge = [c04_ref[k:k + 1, :] + jnp.dot(gwv, m4_ref[k, :, :],
                                        preferred_element_type=jnp.float32)
          for k in range(4)]
    g_t0 = ge[0]
    g_sh[0] = g_sh[0] + jnp.sum(ge[0] * t0, axis=1, keepdims=True)
    for k in range(1, 4):
        g_t0 = g_t0 + ge[k] * (SQ3 * u[:, k - 1:k])
        g_sh[k] = g_sh[k] + jnp.sum(ge[k] * t0, axis=1, keepdims=True)
    g_rw0 = g_t0 * hs0

    def radial_bwd(g_rw, rwp, z1, z2, w3t, w2t, w1t):
        g_cut = jnp.sum(g_rw * rwp, axis=1, keepdims=True)
        g_rwp = g_rw * env
        g_a2 = jnp.dot(g_rwp, w3t, preferred_element_type=jnp.float32)
        g_z2 = g_a2 * _dsilu(z2)
        g_a1 = jnp.dot(g_z2, w2t, preferred_element_type=jnp.float32)
        g_z1 = g_a1 * _dsilu(z1)
        g_ef = jnp.dot(g_z1, w1t, preferred_element_type=jnp.float32)
        return g_ef, g_cut

    g_ef0, g_cut0 = radial_bwd(g_rw0, rwp0, z1_0, z2_0, wr30t_ref[...], wr20t_ref[...], wr10t_ref[...])
    g_ef1, g_cut1 = radial_bwd(g_rw1, rwp1, z1_1, z2_1, wr31t_ref[...], wr21t_ref[...], wr11t_ref[...])
    g_ef = g_ef0 + g_ef1
    g_cut = g_cut0 + g_cut1

    g_env = g_cut + jnp.sum(g_ef * bes, axis=1, keepdims=True)
    g_bes = g_ef * env
    dbes_dr = (BESC * warr * jnp.cos(arg) - bes) / r
    g_r = jnp.sum(g_bes * dbes_dr, axis=1, keepdims=True) + g_env * denv_dr

    lane = lax.broadcasted_iota(jnp.int32, (ps.shape[0], 16), 1)
    gv = jnp.zeros(ps.shape, jnp.float32)
    udot = jnp.zeros((ps.shape[0], 1), jnp.float32)
    g_u = [None, None, None]
    for k in range(3):
        g_u[k] = SQ3 * g_sh[k + 1]
        udot = udot + u[:, k:k + 1] * g_u[k]
    for k in range(3):
        uk = u[:, k:k + 1]
        gvk = g_u[k] / r - uk * udot / r + uk * g_r
        gv = gv + gvk * (lane == k).astype(jnp.float32)
    gv_ref[...] = gv * valid16


def _finish_body(na_ref, e0p_ref, out0_ref, q1p_ref, hsc1_ref, wread1_ref,
                 ch_ref, pos_ref, batch_ref,
                 fr_ref, fs_ref, forces_ref, ad_ref, ge_ref, gd_ref, gb_ref):
    pid = pl.program_id(0)
    out0 = out0_ref[...]
    q1sum = (q1p_ref[0, 0, :, 0:8] + q1p_ref[0, 1, :, 0:8]) * (1.0 / AVG)
    out1 = q1sum + jnp.dot(hsc1_ref[...], wread1_ref[...],
                           preferred_element_type=jnp.float32)
    outs = out0 + out1
    ad_ref[...] = outs
    forces_ref[...] = -(fr_ref[0, 0, :, :] + fr_ref[0, 1, :, :]
                        - fs_ref[0, 0, :, :] - fs_ref[0, 1, :, :])
    ne0 = jnp.dot(na_ref[...], e0p_ref[...], preferred_element_type=jnp.float32)
    lane8 = lax.broadcasted_iota(jnp.int32, (NBLK, 8), 1)
    l0 = (lane8 == 0).astype(jnp.float32)
    epn = ne0 + outs * l0
    onehot = (batch_ref[...] == lane8).astype(jnp.float32)
    cp = ch_ref[...] * pos_ref[...]
    dn = (((0,), (0,)), ((), ()))
    gE = lax.dot_general(onehot, epn, dn, preferred_element_type=jnp.float32)
    gD = lax.dot_general(onehot, outs, dn, preferred_element_type=jnp.float32)
    gB = lax.dot_general(onehot, cp, dn, preferred_element_type=jnp.float32)

    @pl.when(pid == 0)
    def _():
        ge_ref[...] = gE
        gd_ref[...] = gD
        gb_ref[...] = gB

    @pl.when(pid != 0)
    def _():
        ge_ref[...] = ge_ref[...] + gE
        gd_ref[...] = gd_ref[...] + gD
        gb_ref[...] = gb_ref[...] + gB


def _sc_gather_multi(pairs, D):
    P = len(pairs)
    mesh = plsc.VectorSubcoreMesh(core_axis_name="c", subcore_axis_name="s")

    @functools.partial(
        pl.kernel, mesh=mesh,
        compiler_params=pltpu.CompilerParams(use_tc_tiling_on_sc=(D % 128 == 0)),
        out_type=[jax.ShapeDtypeStruct((EP, D), jnp.float32) for _ in range(P)],
        scratch_types=(
            [pltpu.VMEM((NCH, CHUNK), jnp.int32)]
            + [pltpu.VMEM((CHUNK, D), jnp.float32) for _ in range(4)]
            + [pltpu.SemaphoreType.DMA for _ in range(8)]
        ),
    )
    def k(*args):
        tables = args[:P]
        idxs = args[P:2 * P]
        outs = args[2 * P:3 * P]
        idx_v = args[3 * P]
        bufs = args[3 * P + 1:3 * P + 5]
        gs = args[3 * P + 5:3 * P + 9]
        ws = args[3 * P + 9:3 * P + 13]
        c = lax.axis_index("c")
        s = lax.axis_index("s")
        wid = c * NSUB + s

        for p in range(P):
            table, out = tables[p], outs[p]
            pltpu.sync_copy(idxs[p].at[c, s], idx_v)

            def gat_start(j, b):
                pltpu.async_copy(table.at[idx_v.at[j]], bufs[b], gs[b])

            def gat_wait(b):
                pltpu.make_async_copy(table.at[idx_v.at[0]], bufs[b], gs[b]).wait()

            def wb_start(j, b):
                base = (wid * NCH + j) * CHUNK
                pltpu.async_copy(bufs[b], out.at[pl.ds(base, CHUNK)], ws[b])

            def wb_wait(b):
                base = wid * NCH * CHUNK
                pltpu.make_async_copy(bufs[b], out.at[pl.ds(base, CHUNK)], ws[b]).wait()

            for b in range(4):
                gat_start(b, b)

            def body(it, _):
                j = it * 4
                for b in range(4):
                    gat_wait(b)
                    wb_start(j + b, b)
                for b in range(4):
                    wb_wait(b)
                    gat_start(j + 4 + b, b)
                return 0

            lax.fori_loop(0, NCH // 4 - 1, body, 0)
            jl = NCH - 4
            for b in range(4):
                gat_wait(b)
                wb_start(jl + b, b)
            for b in range(4):
                wb_wait(b)

    ins = [p[0] for p in pairs] + [p[1] for p in pairs]
    return list(k(*ins))


def _sc_scatter(data, idx4, K, D):
    mesh = plsc.VectorSubcoreMesh(core_axis_name="c", subcore_axis_name="s")
    zeros = jnp.zeros((NPT, D), jnp.float32)

    @functools.partial(
        pl.kernel, mesh=mesh,
        compiler_params=pltpu.CompilerParams(use_tc_tiling_on_sc=(D % 128 == 0)),
        out_type=jax.ShapeDtypeStruct((K, NCORES, NP, D), jnp.float32),
        scratch_types=(
            [pltpu.VMEM((NCH, CHUNK), jnp.int32)]
            + [pltpu.VMEM((CHUNK, D), jnp.float32) for _ in range(2)]
            + [pltpu.SemaphoreType.DMA for _ in range(4)]
            + [pltpu.VMEM_SHARED((NP, D), jnp.float32)]
        ),
    )
    def k(data_hbm, idx_hbm, z_hbm, out_hbm, idx_v, b0, b1,
          r0, r1, t0, t1, acc_sh):
        bufs = (b0, b1)
        rs = (r0, r1)
        ss = (t0, t1)
        c = lax.axis_index("c")
        s = lax.axis_index("s")
        pltpu.sync_copy(idx_hbm.at[c, s], idx_v)
        base_rows = (c * NSUB + s) * NCH * CHUNK
        for kk in range(K):
            pltpu.sync_copy(z_hbm, acc_sh.at[pl.ds(s * NPT, NPT)])
            plsc.subcore_barrier()

            def rd_start(j, b):
                pltpu.async_copy(data_hbm.at[kk, pl.ds(base_rows + j * CHUNK, CHUNK)], bufs[b], rs[b])

            def rd_wait(b):
                pltpu.make_async_copy(data_hbm.at[kk, pl.ds(base_rows, CHUNK)], bufs[b], rs[b]).wait()

            def scat_start(j, b):
                pltpu.async_copy(bufs[b], acc_sh.at[idx_v.at[j]], ss[b], add=True)

            def scat_wait(b):
                pltpu.make_async_copy(bufs[b], acc_sh.at[idx_v.at[0]], ss[b]).wait()

            for b in range(2):
                rd_start(b, b)

            def body(it, _):
                j = it * 2
                for b in range(2):
                    rd_wait(b)
                    scat_start(j + b, b)
                for b in range(2):
                    scat_wait(b)
                    rd_start(j + 2 + b, b)
                return 0

            lax.fori_loop(0, NCH // 2 - 1, body, 0)
            jl = NCH - 2
            for b in range(2):
                rd_wait(b)
                scat_start(jl + b, b)
            for b in range(2):
                scat_wait(b)
            plsc.subcore_barrier()
            pltpu.sync_copy(acc_sh.at[pl.ds(s * NPT, NPT)],
                            out_hbm.at[kk, c, pl.ds(s * NPT, NPT)])
            plsc.subcore_barrier()

    return k(data, idx4, zeros)


def _sc_scatter_forces(data, ridx4, sidx4):
    mesh = plsc.VectorSubcoreMesh(core_axis_name="c", subcore_axis_name="s")
    zeros = jnp.zeros((NPT, 16), jnp.float32)

    @functools.partial(
        pl.kernel, mesh=mesh,
        compiler_params=pltpu.CompilerParams(use_tc_tiling_on_sc=False),
        out_type=jax.ShapeDtypeStruct((2, NCORES, NP, 16), jnp.float32),
        scratch_types=[
            pltpu.VMEM((NCH, CHUNK), jnp.int32),
            pltpu.VMEM((NCH, CHUNK), jnp.int32),
            pltpu.VMEM((CHUNK, 16), jnp.float32),
            pltpu.VMEM((CHUNK, 16), jnp.float32),
            pltpu.SemaphoreType.DMA,
            pltpu.SemaphoreType.DMA,
            pltpu.VMEM_SHARED((NP, 16), jnp.float32),
            pltpu.VMEM_SHARED((NP, 16), jnp.float32),
        ],
    )
    def k(data_hbm, ridx_hbm, sidx_hbm, z_hbm, out_hbm,
          ridx_v, sidx_v, buf0, buf1, rs0, rs1, accR, accS):
        c = lax.axis_index("c")
        s = lax.axis_index("s")
        pltpu.sync_copy(ridx_hbm.at[c, s], ridx_v)
        pltpu.sync_copy(sidx_hbm.at[c, s], sidx_v)
        base_rows = (c * NSUB + s) * NCH * CHUNK
        pltpu.sync_copy(z_hbm, accR.at[pl.ds(s * NPT, NPT)])
        pltpu.sync_copy(z_hbm, accS.at[pl.ds(s * NPT, NPT)])
        plsc.subcore_barrier()

        def rd_start(j, buf, sem):
            pltpu.async_copy(data_hbm.at[pl.ds(base_rows + j * CHUNK, CHUNK)], buf, sem)

        def rd_wait(buf, sem):
            pltpu.make_async_copy(data_hbm.at[pl.ds(base_rows, CHUNK)], buf, sem).wait()

        def scat(j, buf):
            pltpu.sync_copy(buf, accR.at[ridx_v.at[j]], add=True)
            pltpu.sync_copy(buf, accS.at[sidx_v.at[j]], add=True)

        rd_start(0, buf0, rs0)
        rd_start(1, buf1, rs1)

        def body(it, _):
            j = it * 2
            rd_wait(buf0, rs0)
            scat(j, buf0)
            rd_start(j + 2, buf0, rs0)
            rd_wait(buf1, rs1)
            scat(j + 1, buf1)
            rd_start(j + 3, buf1, rs1)
            return 0

        lax.fori_loop(0, NCH // 2 - 1, body, 0)
        jl = NCH - 2
        rd_wait(buf0, rs0)
        scat(jl, buf0)
        rd_wait(buf1, rs1)
        scat(jl + 1, buf1)
        plsc.subcore_barrier()
        pltpu.sync_copy(accR.at[pl.ds(s * NPT, NPT)], out_hbm.at[0, c, pl.ds(s * NPT, NPT)])
        pltpu.sync_copy(accS.at[pl.ds(s * NPT, NPT)], out_hbm.at[1, c, pl.ds(s * NPT, NPT)])
        plsc.subcore_barrier()

    return k(data, ridx4, sidx4, zeros)


def _full(shape):
    return pl.BlockSpec(shape, lambda i: tuple(0 for _ in shape))


def _tc_node_pre(node_attrs, W_embed, Wmsg_0, Wsc_0, interpret=False):
    grid = (N // NBLK,)
    return pl.pallas_call(
        _node_pre_body,
        grid=grid,
        in_specs=[
            pl.BlockSpec((NBLK, 10), lambda i: (i, 0)),
            _full((10, F)), _full((F, F)), _full((F, FS)),
        ],
        out_specs=[
            pl.BlockSpec((NBLK, F), lambda i: (i, 0)),
            pl.BlockSpec((NBLK, FS), lambda i: (i, 0)),
        ],
        out_shape=[
            jax.ShapeDtypeStruct((N, F), jnp.float32),
            jax.ShapeDtypeStruct((N, FS), jnp.float32),
        ],
        interpret=interpret,
    )(node_attrs, W_embed, Wmsg_0, Wsc_0)


def _tc_edge_fwd(ps, pr, hs0, Wr1, Wr2, Wr3, interpret=False):
    grid = (EP // EB,)
    return pl.pallas_call(
        _edge_fwd_body,
        grid=grid,
        in_specs=[
            pl.BlockSpec((EB, 16), lambda i: (i, 0)),
            pl.BlockSpec((EB, 16), lambda i: (i, 0)),
            pl.BlockSpec((EB, F), lambda i: (i, 0)),
            _full((NB, 64)), _full((64, 64)), _full((64, F)),
        ],
        out_specs=pl.BlockSpec((4, EB, F), lambda i: (0, i, 0)),
        out_shape=jax.ShapeDtypeStruct((4, EP, F), jnp.float32),
        interpret=interpret,
    )(ps, pr, hs0, Wr1, Wr2, Wr3)


def _tc_node_mid(aggP0, Wp0k, hsc0, Wmsg_1, Wsc_1, Wread0p, interpret=False):
    grid = (N // NBLK,)
    return pl.pallas_call(
        _node_mid_body,
        grid=grid,
        in_specs=[
            pl.BlockSpec((4, NCORES, NBLK, F), lambda i: (0, 0, i, 0)),
            _full((4, F, FS)),
            pl.BlockSpec((NBLK, FS), lambda i: (i, 0)),
            _full((FS, F)), _full((FS, FS)), _full((FS, 8)),
        ],
        out_specs=[
            pl.BlockSpec((NBLK, F), lambda i: (i, 0)),
            pl.BlockSpec((NBLK, FS), lambda i: (i, 0)),
            pl.BlockSpec((NBLK, 8), lambda i: (i, 0)),
        ],
        out_shape=[
            jax.ShapeDtypeStruct((N, F), jnp.float32),
            jax.ShapeDtypeStruct((N, FS), jnp.float32),
            jax.ShapeDtypeStruct((N, 8), jnp.float32),
        ],
        interpret=interpret,
    )(aggP0, Wp0k, hsc0, Wmsg_1, Wsc_1, Wread0p)


def _tc_edge_msg1(ps, pr, hs1, Wr1, Wr2, Wr3, G1T, W4kp, interpret=False):
    grid = (EP // EB,)
    return pl.pallas_call(
        _edge_msg1_body,
        grid=grid,
        in_specs=[
            pl.BlockSpec((EB, 16), lambda i: (i, 0)),
            pl.BlockSpec((EB, 16), lambda i: (i, 0)),
            pl.BlockSpec((EB, F), lambda i: (i, 0)),
            _full((NB, 64)), _full((64, 64)), _full((64, F)), _full((4, F)),
            _full((4, F, 16)),
        ],
        out_specs=[
            pl.BlockSpec((EB, 16), lambda i: (i, 0)),
            pl.BlockSpec((EB, F), lambda i: (i, 0)),
        ],
        out_shape=[
            jax.ShapeDtypeStruct((EP, 16), jnp.float32),
            jax.ShapeDtypeStruct((EP, F), jnp.float32),
        ],
        interpret=interpret,
    )(ps, pr, hs1, Wr1, Wr2, Wr3, G1T, W4kp)


def _tc_node_gw(ghWP, interpret=False):
    grid = (N // NBLK,)
    return pl.pallas_call(
        _node_gw_body,
        grid=grid,
        in_specs=[
            pl.BlockSpec((1, NCORES, NBLK, F), lambda i: (0, 0, i, 0)),
        ],
        out_specs=pl.BlockSpec((NBLK, F), lambda i: (i, 0)),
        out_shape=jax.ShapeDtypeStruct((N, F), jnp.float32),
        interpret=interpret,
    )(ghWP)


def _tc_edge_bwd(ps, pr, hs0, hs1, gwv, wpack, M4, C04, interpret=False):
    grid = (EP // EB,)
    (Wr10, Wr20, Wr30, Wr11, Wr21, Wr31,
     Wr10T, Wr20T, Wr30T, Wr11T, Wr21T, Wr31T, G1T) = wpack
    return pl.pallas_call(
        _edge_bwd_body,
        grid=grid,
        in_specs=[
            pl.BlockSpec((EB, 16), lambda i: (i, 0)),
            pl.BlockSpec((EB, 16), lambda i: (i, 0)),
            pl.BlockSpec((EB, F), lambda i: (i, 0)),
            pl.BlockSpec((EB, F), lambda i: (i, 0)),
            pl.BlockSpec((EB, F), lambda i: (i, 0)),
            _full((NB, 64)), _full((64, 64)), _full((64, F)),
            _full((NB, 64)), _full((64, 64)), _full((64, F)),
            _full((64, NB)), _full((64, 64)), _full((F, 64)),
            _full((64, NB)), _full((64, 64)), _full((F, 64)),
            _full((4, F)), _full((4, F, F)), _full((4, F)),
        ],
        out_specs=pl.BlockSpec((EB, 16), lambda i: (i, 0)),
        out_shape=jax.ShapeDtypeStruct((EP, 16), jnp.float32),
        interpret=interpret,
    )(ps, pr, hs0, hs1, gwv,
      Wr10, Wr20, Wr30, Wr11, Wr21, Wr31,
      Wr10T, Wr20T, Wr30T, Wr11T, Wr21T, Wr31T, G1T, M4, C04)


def _tc_finish(node_attrs, E0p, out0c, q1P, hsc1, Wread1p, charges2, pos16,
               batch2, fR, fS, interpret=False):
    grid = (N // NBLK,)
    return pl.pallas_call(
        _finish_body,
        grid=grid,
        in_specs=[
            pl.BlockSpec((NBLK, 10), lambda i: (i, 0)),
            _full((10, 8)),
            pl.BlockSpec((NBLK, 8), lambda i: (i, 0)),
            pl.BlockSpec((1, NCORES, NBLK, 16), lambda i: (0, 0, i, 0)),
            pl.BlockSpec((NBLK, FS), lambda i: (i, 0)),
            _full((FS, 8)),
            pl.BlockSpec((NBLK, 1), lambda i: (i, 0)),
            pl.BlockSpec((NBLK, 16), lambda i: (i, 0)),
            pl.BlockSpec((NBLK, 8), lambda i: (i, 0)),
            pl.BlockSpec((1, NCORES, NBLK, 16), lambda i: (0, 0, i, 0)),
            pl.BlockSpec((1, NCORES, NBLK, 16), lambda i: (0, 0, i, 0)),
        ],
        out_specs=[
            pl.BlockSpec((NBLK, 16), lambda i: (i, 0)),
            pl.BlockSpec((NBLK, 8), lambda i: (i, 0)),
            _full((G, 8)), _full((G, 8)), _full((G, 16)),
        ],
        out_shape=[
            jax.ShapeDtypeStruct((N, 16), jnp.float32),
            jax.ShapeDtypeStruct((N, 8), jnp.float32),
            jax.ShapeDtypeStruct((G, 8), jnp.float32),
            jax.ShapeDtypeStruct((G, 8), jnp.float32),
            jax.ShapeDtypeStruct((G, 16), jnp.float32),
        ],
        interpret=interpret,
    )(node_attrs, E0p, out0c, q1P, hsc1, Wread1p, charges2, pos16, batch2, fR, fS)


def kernel(node_attrs, positions, shifts, charges, edge_index, batch, head, ptr,
           W_embed, E0, Wr1_0, Wr2_0, Wr3_0, Wmsg_0, Wsc_0, Wprod_0, Wread_0,
           Wr1_1, Wr2_1, Wr3_1, Wmsg_1, Wsc_1, Wprod_1, Wread_1):
    f32 = jnp.float32
    sender = edge_index[0]
    recv = edge_index[1]
    pad = EP - E
    sender_p = jnp.pad(sender, (0, pad)).astype(jnp.int32)
    recv_p = jnp.pad(recv, (0, pad)).astype(jnp.int32)
    sidx4 = sender_p.reshape(NCORES, NSUB, NCH, CHUNK)
    ridx4 = recv_p.reshape(NCORES, NSUB, NCH, CHUNK)
    pos16 = jnp.pad(positions.astype(f32), ((0, 0), (0, 13)))
    charges2 = charges.astype(f32)[:, None]
    batch2 = jnp.broadcast_to(batch.astype(jnp.int32)[:, None], (N, 8))

    Wp0k = Wprod_0.reshape(F, 4, FS).transpose(1, 0, 2)
    Wread0p = jnp.pad(Wread_0, ((0, 0), (0, 4)))
    Wread1p = jnp.pad(Wread_1, ((0, 0), (0, 4)))
    E0p = jnp.pad(E0.T, ((0, 0), (0, 7)))
    G1T = (Wprod_1 @ Wread_1[:, 0]).reshape(F, 4).T / AVG
    g1c = Wread_0[:, 0] + Wsc_1 @ Wread_1[:, 0]
    W4kp = jnp.pad((Wprod_1 @ Wread_1).reshape(F, 4, 4).transpose(1, 0, 2),
                   ((0, 0), (0, 0), (0, 12)))
    M4 = jnp.einsum('ef,kge->kfg', Wmsg_1, Wp0k) / AVG
    C04 = jnp.einsum('e,kge->kg', g1c, Wp0k) / AVG
    wpack = (Wr1_0, Wr2_0, Wr3_0, Wr1_1, Wr2_1, Wr3_1,
             Wr1_0.T, Wr2_0.T, Wr3_0.T, Wr1_1.T, Wr2_1.T, Wr3_1.T, G1T)

    hW0, hsc0 = _tc_node_pre(node_attrs, W_embed, Wmsg_0, Wsc_0)
    ps, pr = _sc_gather_multi([(pos16, sidx4), (pos16, ridx4)], 16)
    hs0, = _sc_gather_multi([(hW0, sidx4)], F)
    m0 = _tc_edge_fwd(ps, pr, hs0, Wr1_0, Wr2_0, Wr3_0)
    aggP0 = _sc_scatter(m0, ridx4, 4, F)
    hW1, hsc1, out0c = _tc_node_mid(aggP0, Wp0k, hsc0, Wmsg_1, Wsc_1, Wread0p)
    hs1, = _sc_gather_multi([(hW1, sidx4)], F)
    q1, ghs1 = _tc_edge_msg1(ps, pr, hs1, Wr1_1, Wr2_1, Wr3_1, G1T, W4kp)
    q1P = _sc_scatter(q1[None], ridx4, 1, 16)
    ghWP = _sc_scatter(ghs1[None], sidx4, 1, F)

    gwsum = _tc_node_gw(ghWP)
    gwv, = _sc_gather_multi([(gwsum, ridx4)], F)
    gv16 = _tc_edge_bwd(ps, pr, hs0, hs1, gwv, wpack, M4, C04)
    fRS = _sc_scatter_forces(gv16, ridx4, sidx4)
    fR = fRS[0:1]
    fS = fRS[1:2]

    forces16, ad8, gE, gD, gB = _tc_finish(
        node_attrs, E0p, out0c, q1P, hsc1, Wread1p, charges2, pos16, batch2, fR, fS)
    total_energy = gE[:, 0]
    total_dipole = gD[:, 1:4] + gB[:, 0:3]
    atomic_dipoles = ad8[:, 1:4]
    forces = forces16[:, 0:3]
    return total_energy, forces, total_dipole, atomic_dipoles


# speedup vs baseline: 1.4002x; 1.0242x over previous
import functools

import jax
import jax.numpy as jnp
from jax import lax
from jax.experimental import pallas as pl
from jax.experimental.pallas import tpu as pltpu
from jax.experimental.pallas import tpu_sc as plsc
import numpy as np

N = 10000
E = 160000
F = 128
FS = 512
NB = 8
G = 8
RMAX = 5.0
AVG = 16.0

NCORES = 2
NSUB = 16
NW = NCORES * NSUB
CHUNK = 128
NCH = 40
EP = NW * NCH * CHUNK
EB = 2048
NBLK = 1000
NP = 10240
NPT = NP // NSUB
SQ3 = float(np.sqrt(3.0))
BESC = float(np.sqrt(2.0 / RMAX))


def _silu(z):
    s = jax.nn.sigmoid(z)
    return z * s


def _dsilu(z):
    s = jax.nn.sigmoid(z)
    return s * (1.0 + z * (1.0 - s))


def _geometry(ps, pr):
    vec = pr - ps
    r2 = jnp.sum(vec * vec, axis=1, keepdims=True) + 1e-12
    r = jnp.sqrt(r2)
    u = vec / r
    x = r / RMAX
    x2 = x * x
    x4 = x2 * x2
    x5 = x4 * x
    x6 = x5 * x
    x7 = x6 * x
    inside = x < 1.0
    env = jnp.where(inside, 1.0 - 21.0 * x5 + 35.0 * x6 - 15.0 * x7, 0.0)
    denv_dr = jnp.where(inside, -105.0 * x4 * (1.0 - x) * (1.0 - x), 0.0) * (1.0 / RMAX)
    narr = lax.broadcasted_iota(jnp.int32, (ps.shape[0], NB), 1).astype(jnp.float32) + 1.0
    warr = narr * (np.pi / RMAX)
    arg = r * warr
    bes = BESC * jnp.sin(arg) / r
    ef = bes * env
    return vec, r, u, env, denv_dr, warr, arg, bes, ef


def _radial_fwd(ef, Wr1, Wr2, Wr3):
    z1 = jnp.dot(ef, Wr1, preferred_element_type=jnp.float32)
    a1 = _silu(z1)
    z2 = jnp.dot(a1, Wr2, preferred_element_type=jnp.float32)
    a2 = _silu(z2)
    rwp = jnp.dot(a2, Wr3, preferred_element_type=jnp.float32)
    return z1, z2, rwp


def _node_pre_body(na_ref, wemb_ref, wmsg0_ref, wb1_ref, wb2_ref, wb3_ref,
                   hw0_ref, b1_ref, b2_ref, b3_ref):
    na = na_ref[...]
    h0 = jnp.dot(na, wemb_ref[...], preferred_element_type=jnp.float32)
    hw0_ref[...] = jnp.dot(h0, wmsg0_ref[...], preferred_element_type=jnp.float32)
    b1_ref[...] = jnp.dot(h0, wb1_ref[...], preferred_element_type=jnp.float32)
    b2_ref[...] = jnp.dot(h0, wb2_ref[...], preferred_element_type=jnp.float32)
    b3_ref[...] = jnp.dot(h0, wb3_ref[...], preferred_element_type=jnp.float32)


def _edge_fwd_body(ps_ref, pr_ref, hs0_ref, wr1_ref, wr2_ref, wr3_ref,
                   xa_ref, xb_ref, za_ref, zb_ref):
    ps = ps_ref[...]
    pr = pr_ref[...]
    _, _, u, env, _, _, _, _, ef = _geometry(ps, pr)
    _, _, rwp = _radial_fwd(ef, wr1_ref[...], wr2_ref[...], wr3_ref[...])
    eid = lax.broadcasted_iota(jnp.int32, (EB, F), 0) + pl.program_id(0) * EB
    valid = (eid < E).astype(jnp.float32)
    rw = rwp * env * valid
    t0 = hs0_ref[...] * rw
    za = jnp.dot(t0, xa_ref[0, :, :], preferred_element_type=jnp.float32)
    zb = jnp.dot(t0, xb_ref[0, :, :], preferred_element_type=jnp.float32)
    for k in range(1, 4):
        shk = SQ3 * u[:, k - 1:k]
        za = za + shk * jnp.dot(t0, xa_ref[k, :, :], preferred_element_type=jnp.float32)
        zb = zb + shk * jnp.dot(t0, xb_ref[k, :, :], preferred_element_type=jnp.float32)
    za_ref[...] = za
    zb_ref[...] = zb


def _node_comb_body(p_ref, b_ref, out_ref):
    out_ref[...] = (p_ref[0, 0, :, :] + p_ref[0, 1, :, :]) * (1.0 / AVG) + b_ref[...]


def _edge_msg1_body(ps_ref, pr_ref, hs1_ref, wr1_ref, wr2_ref, wr3_ref, g1t_ref,
                    w4_ref, q1_ref, ghs1_ref):
    ps = ps_ref[...]
    pr = pr_ref[...]
    _, _, u, env, _, _, _, _, ef = _geometry(ps, pr)
    _, _, rwp = _radial_fwd(ef, wr1_ref[...], wr2_ref[...], wr3_ref[...])
    eid = lax.broadcasted_iota(jnp.int32, (EB, F), 0) + pl.program_id(0) * EB
    valid = (eid < E).astype(jnp.float32)
    rw = rwp * env * valid
    hs1 = hs1_ref[...]
    t1 = hs1 * rw
    g1t = g1t_ref[...]
    q1 = jnp.dot(t1, w4_ref[0, :, :], preferred_element_type=jnp.float32)
    g_t1 = g1t[0:1, :]
    for k in range(1, 4):
        shk = SQ3 * u[:, k - 1:k]
        q1 = q1 + shk * jnp.dot(t1, w4_ref[k, :, :], preferred_element_type=jnp.float32)
        g_t1 = g_t1 + shk * g1t[k:k + 1, :]
    q1_ref[...] = q1
    ghs1_ref[...] = g_t1 * rw


def _node_gw_body(ghw_ref, gw_ref):
    gw_ref[...] = ghw_ref[0, 0, :, :] + ghw_ref[0, 1, :, :]


def _edge_bwd_body(ps_ref, pr_ref, hs0_ref, hs1_ref, gwv_ref,
                   wr10_ref, wr20_ref, wr30_ref, wr11_ref, wr21_ref, wr31_ref,
                   wr10t_ref, wr20t_ref, wr30t_ref, wr11t_ref, wr21t_ref, wr31t_ref,
                   g1t_ref, m4_ref, c04_ref, gv_ref):
    ps = ps_ref[...]
    pr = pr_ref[...]
    _, r, u, env, denv_dr, warr, arg, bes, ef = _geometry(ps, pr)
    eidv = lax.broadcasted_iota(jnp.int32, (ps.shape[0], 16), 0) + pl.program_id(0) * ps.shape[0]
    valid16 = (eidv < E).astype(jnp.float32)

    z1_0, z2_0, rwp0 = _radial_fwd(ef, wr10_ref[...], wr20_ref[...], wr30_ref[...])
    z1_1, z2_1, rwp1 = _radial_fwd(ef, wr11_ref[...], wr21_ref[...], wr31_ref[...])
    rw0 = rwp0 * env
    rw1 = rwp1 * env
    hs0 = hs0_ref[...]
    hs1 = hs1_ref[...]
    t0 = hs0 * rw0
    t1 = hs1 * rw1
    g1t = g1t_ref[...]

    g_t1 = g1t[0:1, :]
    for k in range(1, 4):
        g_t1 = g_t1 + (SQ3 * u[:, k - 1:k]) * g1t[k:k + 1, :]
    g_rw1 = g_t1 * hs1
    g_sh = [jnp.sum(t1 * g1t[k:k + 1, :], axis=1, keepdims=True) for k in range(4)]

    gwv = gwv_ref[...]
    ge = [c04_ref[k:k + 1, :] + jnp.dot(gwv, m4_ref[k, :, :],
                                        preferred_element_type=jnp.float32)
          for k in range(4)]
    g_t0 = ge[0]
    g_sh[0] = g_sh[0] + jnp.sum(ge[0] * t0, axis=1, keepdims=True)
    for k in range(1, 4):
        g_t0 = g_t0 + ge[k] * (SQ3 * u[:, k - 1:k])
        g_sh[k] = g_sh[k] + jnp.sum(ge[k] * t0, axis=1, keepdims=True)
    g_rw0 = g_t0 * hs0

    def radial_bwd(g_rw, rwp, z1, z2, w3t, w2t, w1t):
        g_cut = jnp.sum(g_rw * rwp, axis=1, keepdims=True)
        g_rwp = g_rw * env
        g_a2 = jnp.dot(g_rwp, w3t, preferred_element_type=jnp.float32)
        g_z2 = g_a2 * _dsilu(z2)
        g_a1 = jnp.dot(g_z2, w2t, preferred_element_type=jnp.float32)
        g_z1 = g_a1 * _dsilu(z1)
        g_ef = jnp.dot(g_z1, w1t, preferred_element_type=jnp.float32)
        return g_ef, g_cut

    g_ef0, g_cut0 = radial_bwd(g_rw0, rwp0, z1_0, z2_0, wr30t_ref[...], wr20t_ref[...], wr10t_ref[...])
    g_ef1, g_cut1 = radial_bwd(g_rw1, rwp1, z1_1, z2_1, wr31t_ref[...], wr21t_ref[...], wr11t_ref[...])
    g_ef = g_ef0 + g_ef1
    g_cut = g_cut0 + g_cut1

    g_env = g_cut + jnp.sum(g_ef * bes, axis=1, keepdims=True)
    g_bes = g_ef * env
    dbes_dr = (BESC * warr * jnp.cos(arg) - bes) / r
    g_r = jnp.sum(g_bes * dbes_dr, axis=1, keepdims=True) + g_env * denv_dr

    lane = lax.broadcasted_iota(jnp.int32, (ps.shape[0], 16), 1)
    gv = jnp.zeros(ps.shape, jnp.float32)
    udot = jnp.zeros((ps.shape[0], 1), jnp.float32)
    g_u = [None, None, None]
    for k in range(3):
        g_u[k] = SQ3 * g_sh[k + 1]
        udot = udot + u[:, k:k + 1] * g_u[k]
    for k in range(3):
        uk = u[:, k:k + 1]
        gvk = g_u[k] / r - uk * udot / r + uk * g_r
        gv = gv + gvk * (lane == k).astype(jnp.float32)
    gv_ref[...] = gv * valid16


def _finish_body(na_ref, e0p_ref, zbp_ref, b2_ref, b3_ref, q1p_ref,
                 ch_ref, pos_ref, batch_ref,
                 fr_ref, fs_ref, forces_ref, ad_ref, ge_ref, gd_ref, gb_ref):
    pid = pl.program_id(0)
    zb = (zbp_ref[0, 0, :, :] + zbp_ref[0, 1, :, :]) * (1.0 / AVG)
    out0 = zb[:, 8:16] + b3_ref[...]
    q1sum = (q1p_ref[0, 0, :, 0:8] + q1p_ref[0, 1, :, 0:8]) * (1.0 / AVG)
    out1 = q1sum + zb[:, 0:8] + b2_ref[...]
    outs = out0 + out1
    ad_ref[...] = outs
    forces_ref[...] = -(fr_ref[0, 0, :, :] + fr_ref[0, 1, :, :]
                        - fs_ref[0, 0, :, :] - fs_ref[0, 1, :, :])
    ne0 = jnp.dot(na_ref[...], e0p_ref[...], preferred_element_type=jnp.float32)
    lane8 = lax.broadcasted_iota(jnp.int32, (NBLK, 8), 1)
    l0 = (lane8 == 0).astype(jnp.float32)
    epn = ne0 + outs * l0
    onehot = (batch_ref[...] == lane8).astype(jnp.float32)
    cp = ch_ref[...] * pos_ref[...]
    dn = (((0,), (0,)), ((), ()))
    gE = lax.dot_general(onehot, epn, dn, preferred_element_type=jnp.float32)
    gD = lax.dot_general(onehot, outs, dn, preferred_element_type=jnp.float32)
    gB = lax.dot_general(onehot, cp, dn, preferred_element_type=jnp.float32)

    @pl.when(pid == 0)
    def _():
        ge_ref[...] = gE
        gd_ref[...] = gD
        gb_ref[...] = gB

    @pl.when(pid != 0)
    def _():
        ge_ref[...] = ge_ref[...] + gE
        gd_ref[...] = gd_ref[...] + gD
        gb_ref[...] = gb_ref[...] + gB


def _sc_gather_multi(pairs, D):
    P = len(pairs)
    mesh = plsc.VectorSubcoreMesh(core_axis_name="c", subcore_axis_name="s")

    @functools.partial(
        pl.kernel, mesh=mesh,
        compiler_params=pltpu.CompilerParams(use_tc_tiling_on_sc=(D % 128 == 0)),
        out_type=[jax.ShapeDtypeStruct((EP, D), jnp.float32) for _ in range(P)],
        scratch_types=(
            [pltpu.VMEM((NCH, CHUNK), jnp.int32)]
            + [pltpu.VMEM((CHUNK, D), jnp.float32) for _ in range(4)]
            + [pltpu.SemaphoreType.DMA for _ in range(8)]
        ),
    )
    def k(*args):
        tables = args[:P]
        idxs = args[P:2 * P]
        outs = args[2 * P:3 * P]
        idx_v = args[3 * P]
        bufs = args[3 * P + 1:3 * P + 5]
        gs = args[3 * P + 5:3 * P + 9]
        ws = args[3 * P + 9:3 * P + 13]
        c = lax.axis_index("c")
        s = lax.axis_index("s")
        wid = c * NSUB + s

        for p in range(P):
            table, out = tables[p], outs[p]
            pltpu.sync_copy(idxs[p].at[c, s], idx_v)

            def gat_start(j, b):
                pltpu.async_copy(table.at[idx_v.at[j]], bufs[b], gs[b])

            def gat_wait(b):
                pltpu.make_async_copy(table.at[idx_v.at[0]], bufs[b], gs[b]).wait()

            def wb_start(j, b):
                base = (wid * NCH + j) * CHUNK
                pltpu.async_copy(bufs[b], out.at[pl.ds(base, CHUNK)], ws[b])

            def wb_wait(b):
                base = wid * NCH * CHUNK
                pltpu.make_async_copy(bufs[b], out.at[pl.ds(base, CHUNK)], ws[b]).wait()

            for b in range(4):
                gat_start(b, b)

            def body(it, _):
                j = it * 4
                for b in range(4):
                    gat_wait(b)
                    wb_start(j + b, b)
                for b in range(4):
                    wb_wait(b)
                    gat_start(j + 4 + b, b)
                return 0

            lax.fori_loop(0, NCH // 4 - 1, body, 0)
            jl = NCH - 4
            for b in range(4):
                gat_wait(b)
                wb_start(jl + b, b)
            for b in range(4):
                wb_wait(b)

    ins = [p[0] for p in pairs] + [p[1] for p in pairs]
    return list(k(*ins))


def _sc_scatter(data, idx4, K, D):
    mesh = plsc.VectorSubcoreMesh(core_axis_name="c", subcore_axis_name="s")
    zeros = jnp.zeros((NPT, D), jnp.float32)

    @functools.partial(
        pl.kernel, mesh=mesh,
        compiler_params=pltpu.CompilerParams(use_tc_tiling_on_sc=(D % 128 == 0)),
        out_type=jax.ShapeDtypeStruct((K, NCORES, NP, D), jnp.float32),
        scratch_types=(
            [pltpu.VMEM((NCH, CHUNK), jnp.int32)]
            + [pltpu.VMEM((CHUNK, D), jnp.float32) for _ in range(2)]
            + [pltpu.SemaphoreType.DMA for _ in range(4)]
            + [pltpu.VMEM_SHARED((NP, D), jnp.float32)]
        ),
    )
    def k(data_hbm, idx_hbm, z_hbm, out_hbm, idx_v, b0, b1,
          r0, r1, t0, t1, acc_sh):
        bufs = (b0, b1)
        rs = (r0, r1)
        ss = (t0, t1)
        c = lax.axis_index("c")
        s = lax.axis_index("s")
        pltpu.sync_copy(idx_hbm.at[c, s], idx_v)
        base_rows = (c * NSUB + s) * NCH * CHUNK
        for kk in range(K):
            pltpu.sync_copy(z_hbm, acc_sh.at[pl.ds(s * NPT, NPT)])
            plsc.subcore_barrier()

            def rd_start(j, b):
                pltpu.async_copy(data_hbm.at[kk, pl.ds(base_rows + j * CHUNK, CHUNK)], bufs[b], rs[b])

            def rd_wait(b):
                pltpu.make_async_copy(data_hbm.at[kk, pl.ds(base_rows, CHUNK)], bufs[b], rs[b]).wait()

            def scat_start(j, b):
                pltpu.async_copy(bufs[b], acc_sh.at[idx_v.at[j]], ss[b], add=True)

            def scat_wait(b):
                pltpu.make_async_copy(bufs[b], acc_sh.at[idx_v.at[0]], ss[b]).wait()

            for b in range(2):
                rd_start(b, b)

            def body(it, _):
                j = it * 2
                for b in range(2):
                    rd_wait(b)
                    scat_start(j + b, b)
                for b in range(2):
                    scat_wait(b)
                    rd_start(j + 2 + b, b)
                return 0

            lax.fori_loop(0, NCH // 2 - 1, body, 0)
            jl = NCH - 2
            for b in range(2):
                rd_wait(b)
                scat_start(jl + b, b)
            for b in range(2):
                scat_wait(b)
            plsc.subcore_barrier()
            pltpu.sync_copy(acc_sh.at[pl.ds(s * NPT, NPT)],
                            out_hbm.at[kk, c, pl.ds(s * NPT, NPT)])
            plsc.subcore_barrier()

    return k(data, idx4, zeros)


def _sc_scatter_forces(data, ridx4, sidx4):
    mesh = plsc.VectorSubcoreMesh(core_axis_name="c", subcore_axis_name="s")
    zeros = jnp.zeros((NPT, 16), jnp.float32)

    @functools.partial(
        pl.kernel, mesh=mesh,
        compiler_params=pltpu.CompilerParams(use_tc_tiling_on_sc=False),
        out_type=jax.ShapeDtypeStruct((2, NCORES, NP, 16), jnp.float32),
        scratch_types=[
            pltpu.VMEM((NCH, CHUNK), jnp.int32),
            pltpu.VMEM((NCH, CHUNK), jnp.int32),
            pltpu.VMEM((CHUNK, 16), jnp.float32),
            pltpu.VMEM((CHUNK, 16), jnp.float32),
            pltpu.SemaphoreType.DMA,
            pltpu.SemaphoreType.DMA,
            pltpu.SemaphoreType.DMA,
            pltpu.SemaphoreType.DMA,
            pltpu.SemaphoreType.DMA,
            pltpu.SemaphoreType.DMA,
            pltpu.VMEM_SHARED((NP, 16), jnp.float32),
            pltpu.VMEM_SHARED((NP, 16), jnp.float32),
        ],
    )
    def k(data_hbm, ridx_hbm, sidx_hbm, z_hbm, out_hbm,
          ridx_v, sidx_v, buf0, buf1, rs0, rs1, sr0, sr1, ss0, ss1, accR, accS):
        c = lax.axis_index("c")
        s = lax.axis_index("s")
        pltpu.sync_copy(ridx_hbm.at[c, s], ridx_v)
        pltpu.sync_copy(sidx_hbm.at[c, s], sidx_v)
        base_rows = (c * NSUB + s) * NCH * CHUNK
        pltpu.sync_copy(z_hbm, accR.at[pl.ds(s * NPT, NPT)])
        pltpu.sync_copy(z_hbm, accS.at[pl.ds(s * NPT, NPT)])
        plsc.subcore_barrier()

        def rd_start(j, buf, sem):
            pltpu.async_copy(data_hbm.at[pl.ds(base_rows + j * CHUNK, CHUNK)], buf, sem)

        def rd_wait(buf, sem):
            pltpu.make_async_copy(data_hbm.at[pl.ds(base_rows, CHUNK)], buf, sem).wait()

        def scat_start(j, buf, semr, sems):
            pltpu.async_copy(buf, accR.at[ridx_v.at[j]], semr, add=True)
            pltpu.async_copy(buf, accS.at[sidx_v.at[j]], sems, add=True)

        def scat_wait(buf, semr, sems):
            pltpu.make_async_copy(buf, accR.at[ridx_v.at[0]], semr).wait()
            pltpu.make_async_copy(buf, accS.at[sidx_v.at[0]], sems).wait()

        rd_start(0, buf0, rs0)
        rd_start(1, buf1, rs1)

        def body(it, _):
            j = it * 2
            rd_wait(buf0, rs0)
            scat_start(j, buf0, sr0, ss0)
            rd_wait(buf1, rs1)
            scat_start(j + 1, buf1, sr1, ss1)
            scat_wait(buf0, sr0, ss0)
            rd_start(j + 2, buf0, rs0)
            scat_wait(buf1, sr1, ss1)
            rd_start(j + 3, buf1, rs1)
            return 0

        lax.fori_loop(0, NCH // 2 - 1, body, 0)
        jl = NCH - 2
        rd_wait(buf0, rs0)
        scat_start(jl, buf0, sr0, ss0)
        rd_wait(buf1, rs1)
        scat_start(jl + 1, buf1, sr1, ss1)
        scat_wait(buf0, sr0, ss0)
        scat_wait(buf1, sr1, ss1)
        plsc.subcore_barrier()
        pltpu.sync_copy(accR.at[pl.ds(s * NPT, NPT)], out_hbm.at[0, c, pl.ds(s * NPT, NPT)])
        pltpu.sync_copy(accS.at[pl.ds(s * NPT, NPT)], out_hbm.at[1, c, pl.ds(s * NPT, NPT)])
        plsc.subcore_barrier()

    return k(data, ridx4, sidx4, zeros)


def _full(shape):
    return pl.BlockSpec(shape, lambda i: tuple(0 for _ in shape))


def _tc_node_pre(node_attrs, W_embed, Wmsg_0, WB1, WB2, WB3, interpret=False):
    grid = (N // NBLK,)
    return pl.pallas_call(
        _node_pre_body,
        grid=grid,
        in_specs=[
            pl.BlockSpec((NBLK, 10), lambda i: (i, 0)),
            _full((10, F)), _full((F, F)), _full((F, F)),
            _full((F, 8)), _full((F, 8)),
        ],
        out_specs=[
            pl.BlockSpec((NBLK, F), lambda i: (i, 0)),
            pl.BlockSpec((NBLK, F), lambda i: (i, 0)),
            pl.BlockSpec((NBLK, 8), lambda i: (i, 0)),
            pl.BlockSpec((NBLK, 8), lambda i: (i, 0)),
        ],
        out_shape=[
            jax.ShapeDtypeStruct((N, F), jnp.float32),
            jax.ShapeDtypeStruct((N, F), jnp.float32),
            jax.ShapeDtypeStruct((N, 8), jnp.float32),
            jax.ShapeDtypeStruct((N, 8), jnp.float32),
        ],
        interpret=interpret,
    )(node_attrs, W_embed, Wmsg_0, WB1, WB2, WB3)


def _tc_edge_fwd(ps, pr, hs0, Wr1, Wr2, Wr3, XAk, XBk, interpret=False):
    grid = (EP // EB,)
    return pl.pallas_call(
        _edge_fwd_body,
        grid=grid,
        in_specs=[
            pl.BlockSpec((EB, 16), lambda i: (i, 0)),
            pl.BlockSpec((EB, 16), lambda i: (i, 0)),
            pl.BlockSpec((EB, F), lambda i: (i, 0)),
            _full((NB, 64)), _full((64, 64)), _full((64, F)),
            _full((4, F, F)), _full((4, F, 16)),
        ],
        out_specs=[
            pl.BlockSpec((EB, F), lambda i: (i, 0)),
            pl.BlockSpec((EB, 16), lambda i: (i, 0)),
        ],
        out_shape=[
            jax.ShapeDtypeStruct((EP, F), jnp.float32),
            jax.ShapeDtypeStruct((EP, 16), jnp.float32),
        ],
        interpret=interpret,
    )(ps, pr, hs0, Wr1, Wr2, Wr3, XAk, XBk)


def _tc_node_comb(P, B, D, interpret=False):
    grid = (N // NBLK,)
    return pl.pallas_call(
        _node_comb_body,
        grid=grid,
        in_specs=[
            pl.BlockSpec((1, NCORES, NBLK, D), lambda i: (0, 0, i, 0)),
            pl.BlockSpec((NBLK, D), lambda i: (i, 0)),
        ],
        out_specs=pl.BlockSpec((NBLK, D), lambda i: (i, 0)),
        out_shape=jax.ShapeDtypeStruct((N, D), jnp.float32),
        interpret=interpret,
    )(P, B)


def _tc_edge_msg1(ps, pr, hs1, Wr1, Wr2, Wr3, G1T, W4kp, interpret=False):
    grid = (EP // EB,)
    return pl.pallas_call(
        _edge_msg1_body,
        grid=grid,
        in_specs=[
            pl.BlockSpec((EB, 16), lambda i: (i, 0)),
            pl.BlockSpec((EB, 16), lambda i: (i, 0)),
            pl.BlockSpec((EB, F), lambda i: (i, 0)),
            _full((NB, 64)), _full((64, 64)), _full((64, F)), _full((4, F)),
            _full((4, F, 16)),
        ],
        out_specs=[
            pl.BlockSpec((EB, 16), lambda i: (i, 0)),
            pl.BlockSpec((EB, F), lambda i: (i, 0)),
        ],
        out_shape=[
            jax.ShapeDtypeStruct((EP, 16), jnp.float32),
            jax.ShapeDtypeStruct((EP, F), jnp.float32),
        ],
        interpret=interpret,
    )(ps, pr, hs1, Wr1, Wr2, Wr3, G1T, W4kp)


def _tc_node_gw(ghWP, interpret=False):
    grid = (N // NBLK,)
    return pl.pallas_call(
        _node_gw_body,
        grid=grid,
        in_specs=[
            pl.BlockSpec((1, NCORES, NBLK, F), lambda i: (0, 0, i, 0)),
        ],
        out_specs=pl.BlockSpec((NBLK, F), lambda i: (i, 0)),
        out_shape=jax.ShapeDtypeStruct((N, F), jnp.float32),
        interpret=interpret,
    )(ghWP)


def _tc_edge_bwd(ps, pr, hs0, hs1, gwv, wpack, M4, C04, interpret=False):
    grid = (EP // EB,)
    (Wr10, Wr20, Wr30, Wr11, Wr21, Wr31,
     Wr10T, Wr20T, Wr30T, Wr11T, Wr21T, Wr31T, G1T) = wpack
    return pl.pallas_call(
        _edge_bwd_body,
        grid=grid,
        in_specs=[
            pl.BlockSpec((EB, 16), lambda i: (i, 0)),
            pl.BlockSpec((EB, 16), lambda i: (i, 0)),
            pl.BlockSpec((EB, F), lambda i: (i, 0)),
            pl.BlockSpec((EB, F), lambda i: (i, 0)),
            pl.BlockSpec((EB, F), lambda i: (i, 0)),
            _full((NB, 64)), _full((64, 64)), _full((64, F)),
            _full((NB, 64)), _full((64, 64)), _full((64, F)),
            _full((64, NB)), _full((64, 64)), _full((F, 64)),
            _full((64, NB)), _full((64, 64)), _full((F, 64)),
            _full((4, F)), _full((4, F, F)), _full((4, F)),
        ],
        out_specs=pl.BlockSpec((EB, 16), lambda i: (i, 0)),
        out_shape=jax.ShapeDtypeStruct((EP, 16), jnp.float32),
        interpret=interpret,
    )(ps, pr, hs0, hs1, gwv,
      Wr10, Wr20, Wr30, Wr11, Wr21, Wr31,
      Wr10T, Wr20T, Wr30T, Wr11T, Wr21T, Wr31T, G1T, M4, C04)


def _tc_finish(node_attrs, E0p, zbP, B2, B3, q1P, charges2, pos16,
               batch2, fR, fS, interpret=False):
    grid = (N // NBLK,)
    return pl.pallas_call(
        _finish_body,
        grid=grid,
        in_specs=[
            pl.BlockSpec((NBLK, 10), lambda i: (i, 0)),
            _full((10, 8)),
            pl.BlockSpec((1, NCORES, NBLK, 16), lambda i: (0, 0, i, 0)),
            pl.BlockSpec((NBLK, 8), lambda i: (i, 0)),
            pl.BlockSpec((NBLK, 8), lambda i: (i, 0)),
            pl.BlockSpec((1, NCORES, NBLK, 16), lambda i: (0, 0, i, 0)),
            pl.BlockSpec((NBLK, 1), lambda i: (i, 0)),
            pl.BlockSpec((NBLK, 16), lambda i: (i, 0)),
            pl.BlockSpec((NBLK, 8), lambda i: (i, 0)),
            pl.BlockSpec((1, NCORES, NBLK, 16), lambda i: (0, 0, i, 0)),
            pl.BlockSpec((1, NCORES, NBLK, 16), lambda i: (0, 0, i, 0)),
        ],
        out_specs=[
            pl.BlockSpec((NBLK, 16), lambda i: (i, 0)),
            pl.BlockSpec((NBLK, 8), lambda i: (i, 0)),
            _full((G, 8)), _full((G, 8)), _full((G, 16)),
        ],
        out_shape=[
            jax.ShapeDtypeStruct((N, 16), jnp.float32),
            jax.ShapeDtypeStruct((N, 8), jnp.float32),
            jax.ShapeDtypeStruct((G, 8), jnp.float32),
            jax.ShapeDtypeStruct((G, 8), jnp.float32),
            jax.ShapeDtypeStruct((G, 16), jnp.float32),
        ],
        interpret=interpret,
    )(node_attrs, E0p, zbP, B2, B3, q1P, charges2, pos16, batch2, fR, fS)


def kernel(node_attrs, positions, shifts, charges, edge_index, batch, head, ptr,
           W_embed, E0, Wr1_0, Wr2_0, Wr3_0, Wmsg_0, Wsc_0, Wprod_0, Wread_0,
           Wr1_1, Wr2_1, Wr3_1, Wmsg_1, Wsc_1, Wprod_1, Wread_1):
    f32 = jnp.float32
    sender = edge_index[0]
    recv = edge_index[1]
    pad = EP - E
    sender_p = jnp.pad(sender, (0, pad)).astype(jnp.int32)
    recv_p = jnp.pad(recv, (0, pad)).astype(jnp.int32)
    sidx4 = sender_p.reshape(NCORES, NSUB, NCH, CHUNK)
    ridx4 = recv_p.reshape(NCORES, NSUB, NCH, CHUNK)
    pos16 = jnp.pad(positions.astype(f32), ((0, 0), (0, 13)))
    charges2 = charges.astype(f32)[:, None]
    batch2 = jnp.broadcast_to(batch.astype(jnp.int32)[:, None], (N, 8))

    Wp0k = Wprod_0.reshape(F, 4, FS).transpose(1, 0, 2)
    Wread0p = jnp.pad(Wread_0, ((0, 0), (0, 4)))
    Wread1p = jnp.pad(Wread_1, ((0, 0), (0, 4)))
    E0p = jnp.pad(E0.T, ((0, 0), (0, 7)))
    G1T = (Wprod_1 @ Wread_1[:, 0]).reshape(F, 4).T / AVG
    g1c = Wread_0[:, 0] + Wsc_1 @ Wread_1[:, 0]
    W4kp = jnp.pad((Wprod_1 @ Wread_1).reshape(F, 4, 4).transpose(1, 0, 2),
                   ((0, 0), (0, 0), (0, 12)))
    M4 = jnp.einsum('ef,kge->kfg', Wmsg_1, Wp0k) / AVG
    C04 = jnp.einsum('e,kge->kg', g1c, Wp0k) / AVG
    XB_cols = jnp.concatenate([Wsc_1 @ Wread1p, Wread0p], axis=1)
    XAk = jnp.einsum('kge,ef->kgf', Wp0k, Wmsg_1)
    XBk = jnp.einsum('kge,ec->kgc', Wp0k, XB_cols)
    WB1 = Wsc_0 @ Wmsg_1
    WB2 = Wsc_0 @ Wsc_1 @ Wread1p
    WB3 = Wsc_0 @ Wread0p
    wpack = (Wr1_0, Wr2_0, Wr3_0, Wr1_1, Wr2_1, Wr3_1,
             Wr1_0.T, Wr2_0.T, Wr3_0.T, Wr1_1.T, Wr2_1.T, Wr3_1.T, G1T)

    hW0, B1n, B2n, B3n = _tc_node_pre(node_attrs, W_embed, Wmsg_0, WB1, WB2, WB3)
    ps, pr = _sc_gather_multi([(pos16, sidx4), (pos16, ridx4)], 16)
    hs0, = _sc_gather_multi([(hW0, sidx4)], F)
    za, zb = _tc_edge_fwd(ps, pr, hs0, Wr1_0, Wr2_0, Wr3_0, XAk, XBk)
    zaP = _sc_scatter(za[None], ridx4, 1, F)
    zbP = _sc_scatter(zb[None], ridx4, 1, 16)
    hW1 = _tc_node_comb(zaP, B1n, F)
    hs1, = _sc_gather_multi([(hW1, sidx4)], F)
    q1, ghs1 = _tc_edge_msg1(ps, pr, hs1, Wr1_1, Wr2_1, Wr3_1, G1T, W4kp)
    q1P = _sc_scatter(q1[None], ridx4, 1, 16)
    ghWP = _sc_scatter(ghs1[None], sidx4, 1, F)

    gwsum = _tc_node_gw(ghWP)
    gwv, = _sc_gather_multi([(gwsum, ridx4)], F)
    gv16 = _tc_edge_bwd(ps, pr, hs0, hs1, gwv, wpack, M4, C04)
    fRS = _sc_scatter_forces(gv16, ridx4, sidx4)
    fR = fRS[0:1]
    fS = fRS[1:2]

    forces16, ad8, gE, gD, gB = _tc_finish(
        node_attrs, E0p, zbP, B2n, B3n, q1P, charges2, pos16, batch2, fR, fS)
    total_energy = gE[:, 0]
    total_dipole = gD[:, 1:4] + gB[:, 0:3]
    atomic_dipoles = ad8[:, 1:4]
    forces = forces16[:, 0:3]
    return total_energy, forces, total_dipole, atomic_dipoles
